# baseline (speedup 1.0000x reference)
import functools

import jax
import jax.numpy as jnp
from jax import lax
from jax.experimental import pallas as pl
from jax.experimental.pallas import tpu as pltpu
from jax.experimental.pallas import tpu_sc as plsc

D = 128
B_TOTAL = 4096 * 200
NC, NS = 2, 16
NW = NC * NS
BPW = B_TOTAL // NW
SUB = 128
C = 128
NCH = BPW // C
NB = 6
ROWS_X = B_TOTAL // SUB

_mesh = plsc.VectorSubcoreMesh(core_axis_name="c", subcore_axis_name="s")


@functools.partial(
    pl.kernel,
    mesh=_mesh,
    out_type=jax.ShapeDtypeStruct((B_TOTAL, D), jnp.float32),
    scratch_types=(
        [pltpu.VMEM((BPW // SUB, SUB), jnp.int32)]
        + [pltpu.VMEM((C, D), jnp.float32)] * NB
        + [pltpu.SemaphoreType.DMA] * (2 * NB)
    ),
)
def _emb_lookup(x_hbm, tab_hbm, out_hbm, idx_all, *bufs):
    rows = bufs[:NB]
    gsem = bufs[NB:2 * NB]
    osem = bufs[2 * NB:]

    wid = lax.axis_index("s") * NC + lax.axis_index("c")
    rx = wid * (BPW // SUB)

    pltpu.sync_copy(x_hbm.at[pl.ds(rx, BPW // SUB)], idx_all)

    def fire_gather(g, b):
        pltpu.async_copy(tab_hbm.at[idx_all.at[g]], rows[b], gsem[b])

    def drain_gather(g, b):
        pltpu.make_async_copy(tab_hbm.at[idx_all.at[g]], rows[b], gsem[b]).wait()

    def fire_store(g, b):
        pltpu.async_copy(rows[b], out_hbm.at[pl.ds((rx + g) * C, C)], osem[b])

    def drain_store(g, b):
        pltpu.make_async_copy(
            rows[b], out_hbm.at[pl.ds((rx + g) * C, C)], osem[b]
        ).wait()

    def sched(g, b, do_ds=True, do_fg=True):
        drain_gather(g - 2, (b + NB - 2) % NB)
        fire_store(g - 2, (b + NB - 2) % NB)
        if do_ds:
            drain_store(g - 4, (b + NB - 4) % NB)
        if do_fg:
            fire_gather(g, b)

    fire_gather(0, 0)
    fire_gather(1, 1)
    sched(2, 2, do_ds=False)
    sched(3, 3, do_ds=False)
    sched(4, 4)
    sched(5, 5)

    def body(p, carry):
        g0 = NB * p
        for k in range(NB):
            sched(g0 + k, k)
        return carry

    lax.fori_loop(1, (NCH - 2) // NB, body, 0)

    sched(NCH - 2, 0)
    sched(NCH - 1, 1)
    drain_gather(NCH - 2, 0)
    fire_store(NCH - 2, 0)
    drain_store(NCH - 4, 4)
    drain_gather(NCH - 1, 1)
    fire_store(NCH - 1, 1)
    drain_store(NCH - 3, 5)
    drain_store(NCH - 2, 0)
    drain_store(NCH - 1, 1)


def kernel(x, table):
    xf = x.astype(jnp.int32).reshape(ROWS_X, SUB)
    out = _emb_lookup(xf, table)
    return out.reshape(x.shape[0], x.shape[1], D)


# speedup vs baseline: 1.0069x; 1.0069x over previous
import functools

import jax
import jax.numpy as jnp
from jax import lax
from jax.experimental import pallas as pl
from jax.experimental.pallas import tpu as pltpu
from jax.experimental.pallas import tpu_sc as plsc

D = 128
B_TOTAL = 4096 * 200
NC, NS = 2, 16
NW = NC * NS
BPW = B_TOTAL // NW
SUB = 128
C = 256
NSUB = C // SUB
NCH = BPW // C
ROWS_X = B_TOTAL // SUB

_mesh = plsc.VectorSubcoreMesh(core_axis_name="c", subcore_axis_name="s")


@functools.partial(
    pl.kernel,
    mesh=_mesh,
    out_type=jax.ShapeDtypeStruct((B_TOTAL, D), jnp.float32),
    scratch_types=[
        pltpu.VMEM((BPW // SUB, SUB), jnp.int32),
        pltpu.VMEM((C, D), jnp.float32),
        pltpu.VMEM((C, D), jnp.float32),
        pltpu.VMEM((C, D), jnp.float32),
        pltpu.SemaphoreType.DMA,
        pltpu.SemaphoreType.DMA,
        pltpu.SemaphoreType.DMA,
        pltpu.SemaphoreType.DMA,
        pltpu.SemaphoreType.DMA,
        pltpu.SemaphoreType.DMA,
    ],
)
def _emb_lookup(x_hbm, tab_hbm, out_hbm, idx_all,
                rows0, rows1, rows2, gsem0, gsem1, gsem2, osem0, osem1, osem2):
    wid = lax.axis_index("s") * NC + lax.axis_index("c")
    rx = wid * (BPW // SUB)

    rows = (rows0, rows1, rows2)
    gsem = (gsem0, gsem1, gsem2)
    osem = (osem0, osem1, osem2)

    pltpu.sync_copy(x_hbm.at[pl.ds(rx, BPW // SUB)], idx_all)

    def fire_gather(g, b):
        for j in range(NSUB):
            pltpu.async_copy(
                tab_hbm.at[idx_all.at[g * NSUB + j]],
                rows[b].at[pl.ds(j * SUB, SUB)], gsem[b]
            )

    def drain_gather(g, b):
        for j in range(NSUB):
            pltpu.make_async_copy(
                tab_hbm.at[idx_all.at[g * NSUB + j]],
                rows[b].at[pl.ds(j * SUB, SUB)], gsem[b]
            ).wait()

    def fire_store(g, b):
        pltpu.async_copy(rows[b], out_hbm.at[pl.ds((rx + g * NSUB) * SUB, C)], osem[b])

    def drain_store(g, b):
        pltpu.make_async_copy(
            rows[b], out_hbm.at[pl.ds((rx + g * NSUB) * SUB, C)], osem[b]
        ).wait()

    def ring_iter(g, b, pb, first=False):
        drain_gather(g - 2, pb)
        fire_store(g - 2, pb)
        if not first:
            drain_store(g - 3, b)
        fire_gather(g, b)

    fire_gather(0, 0)
    fire_gather(1, 1)
    ring_iter(2, 2, 0, first=True)

    def body(p, carry):
        g = 3 * p
        ring_iter(g, 0, 1)
        ring_iter(g + 1, 1, 2)
        ring_iter(g + 2, 2, 0)
        return carry

    lax.fori_loop(1, (NCH - 4) // 3 + 1, body, 0)

    ring_iter(NCH - 1, 0, 1)
    drain_gather(NCH - 2, 2)
    fire_store(NCH - 2, 2)
    drain_gather(NCH - 1, 0)
    fire_store(NCH - 1, 0)
    drain_store(NCH - 3, 1)
    drain_store(NCH - 2, 2)
    drain_store(NCH - 1, 0)


def kernel(x, table):
    xf = x.astype(jnp.int32).reshape(ROWS_X, SUB)
    out = _emb_lookup(xf, table)
    return out.reshape(x.shape[0], x.shape[1], D)
